# Initial kernel scaffold; baseline (speedup 1.0000x reference)
import jax
import jax.numpy as jnp
from jax.experimental import pallas as pl


def kernel(x, edge_index, edge_basis, W_pre, b_pre, W_rad, b_rad, W_post, b_post):
    raise NotImplementedError("write your pallas kernel here")


# speedup vs baseline: 2.6063x; 2.6063x over previous
import functools

import jax
import jax.numpy as jnp
from jax import lax
from jax.experimental import pallas as pl
from jax.experimental.pallas import tpu as pltpu
from jax.experimental.pallas import tpu_sc as plsc

N_NODES = 10000
N_EDGES = 320000
D_IN = 128
D_RADIAL = 16
D = 128

NC = 2
NS = 16
E_CORE = N_EDGES // NC
E_TILE = E_CORE // NS
CH = 80
NCHUNK = E_TILE // CH
N_TILE = 624
N_TAIL = N_NODES - N_TILE * NS


def _matmul_bias(x, W, b, block_m=None, silu=False):
    M, K = x.shape
    _, N = W.shape
    b2 = b.reshape(1, N)

    def body(x_ref, w_ref, b_ref, o_ref):
        y = jnp.dot(x_ref[...], w_ref[...],
                    preferred_element_type=jnp.float32) + b_ref[...]
        if silu:
            y = y * jax.nn.sigmoid(y)
        o_ref[...] = y

    if block_m is None:
        return pl.pallas_call(
            body, out_shape=jax.ShapeDtypeStruct((M, N), jnp.float32),
        )(x, W, b2)
    return pl.pallas_call(
        body,
        grid=(M // block_m,),
        in_specs=[pl.BlockSpec((block_m, K), lambda i: (i, 0)),
                  pl.BlockSpec((K, N), lambda i: (0, 0)),
                  pl.BlockSpec((1, N), lambda i: (0, 0))],
        out_specs=pl.BlockSpec((block_m, N), lambda i: (i, 0)),
        out_shape=jax.ShapeDtypeStruct((M, N), jnp.float32),
    )(x, W, b2)


def _post_call(parts, W, b):
    b2 = b.reshape(1, D)

    def body(p_ref, w_ref, b_ref, o_ref):
        h = p_ref[0] + p_ref[1]
        y = jnp.dot(h, w_ref[...], preferred_element_type=jnp.float32) + b_ref[...]
        o_ref[...] = y * jax.nn.sigmoid(y)

    return pl.pallas_call(
        body, out_shape=jax.ShapeDtypeStruct((N_NODES, D), jnp.float32),
    )(parts, W, b2)


def _edge_call(hv, filt, src, dst):
    mesh = plsc.VectorSubcoreMesh(core_axis_name="c", subcore_axis_name="s")

    @functools.partial(
        pl.kernel,
        out_type=jax.ShapeDtypeStruct((NC, N_NODES, D), jnp.float32),
        mesh=mesh,
        scratch_types=[
            pltpu.VMEM((CH,), jnp.int32),
            pltpu.VMEM((CH,), jnp.int32),
            pltpu.VMEM((CH, D), jnp.float32),
            pltpu.VMEM((CH, D), jnp.float32),
            pltpu.VMEM((16, D), jnp.float32),
            pltpu.VMEM_SHARED((N_NODES, D), jnp.float32),
            pltpu.SemaphoreType.DMA,
        ],
    )
    def k(hv_hbm, filt_hbm, src_hbm, dst_hbm, out_hbm,
          idx_s, idx_d, filt_v, rows_v, zbuf, h_sh, sem):
        cid = lax.axis_index("c")
        sid = lax.axis_index("s")

        zero = jnp.zeros((16,), jnp.float32)

        def zrow(i, carry):
            for j in range(8):
                zbuf[i, pl.ds(j * 16, 16)] = zero
            return carry

        lax.fori_loop(0, 16, zrow, 0)
        rbase = sid * N_TILE

        def zcp(r, carry):
            pltpu.sync_copy(zbuf, h_sh.at[pl.ds(rbase + r * 16, 16)])
            return carry

        lax.fori_loop(0, N_TILE // 16, zcp, 0)

        @pl.when(sid == 0)
        def _zero_tail():
            pltpu.sync_copy(zbuf, h_sh.at[pl.ds(N_TILE * NS, N_TAIL)])

        plsc.subcore_barrier()

        ebase = cid * E_CORE + sid * E_TILE

        def chunk(ci, carry):
            off = ebase + ci * CH
            pltpu.sync_copy(src_hbm.at[pl.ds(off, CH)], idx_s)
            pltpu.sync_copy(dst_hbm.at[pl.ds(off, CH)], idx_d)
            pltpu.sync_copy(filt_hbm.at[pl.ds(off, CH)], filt_v)
            pltpu.async_copy(hv_hbm.at[idx_s], rows_v, sem).wait()

            def mrow(e, c2):
                for j in range(8):
                    s_ = pl.ds(j * 16, 16)
                    rows_v[e, s_] = rows_v[e, s_] * filt_v[e, s_]
                return c2

            lax.fori_loop(0, CH, mrow, 0)
            pltpu.sync_copy(rows_v, h_sh.at[idx_d], add=True)
            return carry

        lax.fori_loop(0, NCHUNK, chunk, 0)
        plsc.subcore_barrier()

        rows = pl.ds(rbase, N_TILE)
        pltpu.sync_copy(h_sh.at[rows], out_hbm.at[cid, rows])

        @pl.when(sid == 0)
        def _write_tail():
            tail = pl.ds(N_TILE * NS, N_TAIL)
            pltpu.sync_copy(h_sh.at[tail], out_hbm.at[cid, tail])

    return k(hv, filt, src, dst)


def kernel(x, edge_index, edge_basis, W_pre, b_pre, W_rad, b_rad, W_post, b_post):
    src = edge_index[0]
    dst = edge_index[1]
    hv = _matmul_bias(x, W_pre, b_pre)
    filt = _matmul_bias(edge_basis, W_rad, b_rad, block_m=10000)
    parts = _edge_call(hv, filt, src, dst)
    return _post_call(parts, W_post, b_post)


# speedup vs baseline: 4.4338x; 1.7012x over previous
import functools

import jax
import jax.numpy as jnp
from jax import lax
from jax.experimental import pallas as pl
from jax.experimental.pallas import tpu as pltpu
from jax.experimental.pallas import tpu_sc as plsc

N_NODES = 10000
N_EDGES = 320000
D_IN = 128
D_RADIAL = 16
D = 128

NC = 2
NS = 16
E_CORE = N_EDGES // NC
E_TILE = E_CORE // NS
CH = 80
NCHUNK = E_TILE // CH
N_TILE = 624
N_TAIL = N_NODES - N_TILE * NS


def _matmul_bias(x, W, b, block_m=None, silu=False):
    M, K = x.shape
    _, N = W.shape
    b2 = b.reshape(1, N)

    def body(x_ref, w_ref, b_ref, o_ref):
        y = jnp.dot(x_ref[...], w_ref[...],
                    preferred_element_type=jnp.float32) + b_ref[...]
        if silu:
            y = y * jax.nn.sigmoid(y)
        o_ref[...] = y

    if block_m is None:
        return pl.pallas_call(
            body, out_shape=jax.ShapeDtypeStruct((M, N), jnp.float32),
        )(x, W, b2)
    return pl.pallas_call(
        body,
        grid=(M // block_m,),
        in_specs=[pl.BlockSpec((block_m, K), lambda i: (i, 0)),
                  pl.BlockSpec((K, N), lambda i: (0, 0)),
                  pl.BlockSpec((1, N), lambda i: (0, 0))],
        out_specs=pl.BlockSpec((block_m, N), lambda i: (i, 0)),
        out_shape=jax.ShapeDtypeStruct((M, N), jnp.float32),
    )(x, W, b2)


def _post_call(parts, W, b):
    b2 = b.reshape(1, D)

    def body(p_ref, w_ref, b_ref, o_ref):
        h = p_ref[0] + p_ref[1]
        y = jnp.dot(h, w_ref[...], preferred_element_type=jnp.float32) + b_ref[...]
        o_ref[...] = y * jax.nn.sigmoid(y)

    return pl.pallas_call(
        body, out_shape=jax.ShapeDtypeStruct((N_NODES, D), jnp.float32),
    )(parts, W, b2)


def _edge_call(hv, filt, src, dst):
    mesh = plsc.VectorSubcoreMesh(core_axis_name="c", subcore_axis_name="s")

    @functools.partial(
        pl.kernel,
        out_type=jax.ShapeDtypeStruct((NC, N_NODES, D), jnp.float32),
        mesh=mesh,
        scratch_types=[
            [pltpu.VMEM((CH,), jnp.int32)] * 2,
            [pltpu.VMEM((CH,), jnp.int32)] * 2,
            [pltpu.VMEM((CH,), jnp.int32)] * 2,
            [pltpu.VMEM((CH, D), jnp.float32)] * 2,
            [pltpu.VMEM((CH, D), jnp.float32)] * 2,
            pltpu.VMEM((16, D), jnp.float32),
            pltpu.VMEM_SHARED((N_NODES, D), jnp.float32),
            [pltpu.SemaphoreType.DMA] * 2,
            [pltpu.SemaphoreType.DMA] * 2,
            [pltpu.SemaphoreType.DMA] * 2,
        ],
    )
    def k(hv_hbm, filt_hbm, src_hbm, dst_hbm, out_hbm,
          idx_s, idx_d, scidx, filt_v, rows_v, zbuf, h_sh,
          semA, semG, semS):
        cid = lax.axis_index("c")
        sid = lax.axis_index("s")

        zero = jnp.zeros((16,), jnp.float32)

        def zrow(i, carry):
            for j in range(8):
                zbuf[i, pl.ds(j * 16, 16)] = zero
            return carry

        lax.fori_loop(0, 16, zrow, 0)
        rbase = sid * N_TILE

        def zcp(r, carry):
            pltpu.sync_copy(zbuf, h_sh.at[pl.ds(rbase + r * 16, 16)])
            return carry

        lax.fori_loop(0, N_TILE // 16, zcp, 0)

        @pl.when(sid == 0)
        def _zero_tail():
            pltpu.sync_copy(zbuf, h_sh.at[pl.ds(N_TILE * NS, N_TAIL)])

        plsc.subcore_barrier()

        ebase = cid * E_CORE + sid * E_TILE

        def start_loads(ci, b):
            off = ebase + ci * CH
            pltpu.async_copy(src_hbm.at[pl.ds(off, CH)], idx_s[b], semA[b])
            pltpu.async_copy(dst_hbm.at[pl.ds(off, CH)], idx_d[b], semA[b])
            pltpu.async_copy(filt_hbm.at[pl.ds(off, CH)], filt_v[b], semA[b])

        def wait_loads(b):
            pltpu.make_async_copy(src_hbm.at[pl.ds(0, CH)], idx_s[b], semA[b]).wait()
            pltpu.make_async_copy(dst_hbm.at[pl.ds(0, CH)], idx_d[b], semA[b]).wait()
            pltpu.make_async_copy(filt_hbm.at[pl.ds(0, CH)], filt_v[b], semA[b]).wait()

        def start_gather(b):
            pltpu.async_copy(hv_hbm.at[idx_s[b]], rows_v[b], semG[b])

        def wait_gather(b):
            pltpu.make_async_copy(hv_hbm.at[idx_s[b]], rows_v[b], semG[b]).wait()

        def mul_stage(b):
            def mrow(e, c2):
                for j in range(8):
                    s_ = pl.ds(j * 16, 16)
                    rows_v[b][e, s_] = rows_v[b][e, s_] * filt_v[b][e, s_]
                return c2

            lax.fori_loop(0, CH, mrow, 0)
            for j in range(CH // 16):
                s_ = pl.ds(j * 16, 16)
                scidx[b][s_] = idx_d[b][s_]

        def start_scatter(b):
            pltpu.async_copy(rows_v[b], h_sh.at[scidx[b]], semS[b], add=True)

        def wait_scatter(b):
            pltpu.make_async_copy(rows_v[b], h_sh.at[scidx[b]], semS[b]).wait()

        start_loads(0, 0)
        start_loads(1, 1)
        wait_loads(0)
        start_gather(0)
        wait_loads(1)
        start_gather(1)
        wait_gather(0)
        mul_stage(0)
        start_scatter(0)
        start_loads(2, 0)
        wait_gather(1)
        mul_stage(1)
        start_scatter(1)
        start_loads(3, 1)

        def body2(kk, carry):
            c0 = 2 * kk
            wait_loads(0)
            wait_scatter(0)
            start_gather(0)
            wait_loads(1)
            wait_scatter(1)
            start_gather(1)
            wait_gather(0)
            mul_stage(0)
            start_scatter(0)
            start_loads(c0 + 2, 0)
            wait_gather(1)
            mul_stage(1)
            start_scatter(1)

            @pl.when(c0 + 3 < NCHUNK)
            def _():
                start_loads(c0 + 3, 1)

            return carry

        lax.fori_loop(1, (NCHUNK - 1) // 2, body2, 0)

        wait_loads(0)
        wait_scatter(0)
        start_gather(0)
        wait_gather(0)
        mul_stage(0)
        start_scatter(0)
        wait_scatter(0)
        wait_scatter(1)
        plsc.subcore_barrier()

        rows = pl.ds(rbase, N_TILE)
        pltpu.sync_copy(h_sh.at[rows], out_hbm.at[cid, rows])

        @pl.when(sid == 0)
        def _write_tail():
            tail = pl.ds(N_TILE * NS, N_TAIL)
            pltpu.sync_copy(h_sh.at[tail], out_hbm.at[cid, tail])

    return k(hv, filt, src, dst)


def kernel(x, edge_index, edge_basis, W_pre, b_pre, W_rad, b_rad, W_post, b_post):
    src = edge_index[0]
    dst = edge_index[1]
    hv = _matmul_bias(x, W_pre, b_pre)
    filt = _matmul_bias(edge_basis, W_rad, b_rad, block_m=10000)
    parts = _edge_call(hv, filt, src, dst)
    return _post_call(parts, W_post, b_post)


# speedup vs baseline: 4.6298x; 1.0442x over previous
import functools

import jax
import jax.numpy as jnp
from jax import lax
from jax.experimental import pallas as pl
from jax.experimental.pallas import tpu as pltpu
from jax.experimental.pallas import tpu_sc as plsc

N_NODES = 10000
N_EDGES = 320000
D_IN = 128
D_RADIAL = 16
D = 128

NC = 2
NS = 16
E_CORE = N_EDGES // NC
E_TILE = E_CORE // NS
CH = 80
NCHUNK = E_TILE // CH
N_TILE = 624
N_TAIL = N_NODES - N_TILE * NS


def _matmul_bias(x, W, b):
    M, K = x.shape
    _, N = W.shape
    b2 = b.reshape(1, N)

    def body(x_ref, w_ref, b_ref, o_ref):
        o_ref[...] = jnp.dot(x_ref[...], w_ref[...],
                             preferred_element_type=jnp.float32) + b_ref[...]

    return pl.pallas_call(
        body, out_shape=jax.ShapeDtypeStruct((M, N), jnp.float32),
    )(x, W, b2)


def _radial_call(eb8, W_big, b_big, block_m=2000):
    M8 = eb8.shape[0]
    b2 = b_big.reshape(1, 8 * D)

    def body(e_ref, w_ref, b_ref, o_ref):
        y = jnp.dot(e_ref[...], w_ref[...],
                    preferred_element_type=jnp.float32) + b_ref[...]
        o_ref[...] = y.reshape(block_m * 8, D)

    return pl.pallas_call(
        body,
        grid=(M8 // block_m,),
        in_specs=[pl.BlockSpec((block_m, D), lambda i: (i, 0)),
                  pl.BlockSpec((D, 8 * D), lambda i: (0, 0)),
                  pl.BlockSpec((1, 8 * D), lambda i: (0, 0))],
        out_specs=pl.BlockSpec((block_m * 8, D), lambda i: (i, 0)),
        out_shape=jax.ShapeDtypeStruct((N_EDGES, D), jnp.float32),
    )(eb8, W_big, b2)


def _post_call(parts, W, b):
    b2 = b.reshape(1, D)

    def body(p_ref, w_ref, b_ref, o_ref):
        h = p_ref[0] + p_ref[1]
        y = jnp.dot(h, w_ref[...], preferred_element_type=jnp.float32) + b_ref[...]
        o_ref[...] = y * jax.nn.sigmoid(y)

    return pl.pallas_call(
        body, out_shape=jax.ShapeDtypeStruct((N_NODES, D), jnp.float32),
    )(parts, W, b2)


def _edge_call(hv, filt, src, dst):
    mesh = plsc.VectorSubcoreMesh(core_axis_name="c", subcore_axis_name="s")

    @functools.partial(
        pl.kernel,
        out_type=jax.ShapeDtypeStruct((NC, N_NODES, D), jnp.float32),
        mesh=mesh,
        scratch_types=[
            [pltpu.VMEM((CH,), jnp.int32)] * 2,
            [pltpu.VMEM((CH,), jnp.int32)] * 2,
            [pltpu.VMEM((CH,), jnp.int32)] * 2,
            [pltpu.VMEM((CH, D), jnp.float32)] * 2,
            [pltpu.VMEM((CH, D), jnp.float32)] * 2,
            pltpu.VMEM((16, D), jnp.float32),
            pltpu.VMEM_SHARED((N_NODES, D), jnp.float32),
            [pltpu.SemaphoreType.DMA] * 2,
            [pltpu.SemaphoreType.DMA] * 2,
            [pltpu.SemaphoreType.DMA] * 2,
        ],
    )
    def k(hv_hbm, filt_hbm, src_hbm, dst_hbm, out_hbm,
          idx_s, idx_d, scidx, filt_v, rows_v, zbuf, h_sh,
          semA, semG, semS):
        cid = lax.axis_index("c")
        sid = lax.axis_index("s")

        zero = jnp.zeros((16,), jnp.float32)

        def zrow(i, carry):
            for j in range(8):
                zbuf[i, pl.ds(j * 16, 16)] = zero
            return carry

        lax.fori_loop(0, 16, zrow, 0)
        rbase = sid * N_TILE

        def zcp(r, carry):
            pltpu.sync_copy(zbuf, h_sh.at[pl.ds(rbase + r * 16, 16)])
            return carry

        lax.fori_loop(0, N_TILE // 16, zcp, 0)

        @pl.when(sid == 0)
        def _zero_tail():
            pltpu.sync_copy(zbuf, h_sh.at[pl.ds(N_TILE * NS, N_TAIL)])

        plsc.subcore_barrier()

        ebase = cid * E_CORE + sid * E_TILE

        def start_loads(ci, b):
            off = ebase + ci * CH
            pltpu.async_copy(src_hbm.at[pl.ds(off, CH)], idx_s[b], semA[b])
            pltpu.async_copy(dst_hbm.at[pl.ds(off, CH)], idx_d[b], semA[b])
            pltpu.async_copy(filt_hbm.at[pl.ds(off, CH)], filt_v[b], semA[b])

        def wait_loads(b):
            pltpu.make_async_copy(src_hbm.at[pl.ds(0, CH)], idx_s[b], semA[b]).wait()
            pltpu.make_async_copy(dst_hbm.at[pl.ds(0, CH)], idx_d[b], semA[b]).wait()
            pltpu.make_async_copy(filt_hbm.at[pl.ds(0, CH)], filt_v[b], semA[b]).wait()

        def start_gather(b):
            pltpu.async_copy(hv_hbm.at[idx_s[b]], rows_v[b], semG[b])

        def wait_gather(b):
            pltpu.make_async_copy(hv_hbm.at[idx_s[b]], rows_v[b], semG[b]).wait()

        def mul_stage(b):
            def mrow(e, c2):
                for j in range(8):
                    s_ = pl.ds(j * 16, 16)
                    rows_v[b][e, s_] = rows_v[b][e, s_] * filt_v[b][e, s_]
                return c2

            lax.fori_loop(0, CH, mrow, 0)
            for j in range(CH // 16):
                s_ = pl.ds(j * 16, 16)
                scidx[b][s_] = idx_d[b][s_]

        def start_scatter(b):
            pltpu.async_copy(rows_v[b], h_sh.at[scidx[b]], semS[b], add=True)

        def wait_scatter(b):
            pltpu.make_async_copy(rows_v[b], h_sh.at[scidx[b]], semS[b]).wait()

        start_loads(0, 0)
        start_loads(1, 1)
        wait_loads(0)
        start_gather(0)
        wait_loads(1)
        start_gather(1)
        wait_gather(0)
        mul_stage(0)
        start_scatter(0)
        start_loads(2, 0)
        wait_gather(1)
        mul_stage(1)
        start_scatter(1)
        start_loads(3, 1)

        def body2(kk, carry):
            c0 = 2 * kk
            wait_loads(0)
            wait_scatter(0)
            start_gather(0)
            wait_loads(1)
            wait_scatter(1)
            start_gather(1)
            wait_gather(0)
            mul_stage(0)
            start_scatter(0)
            start_loads(c0 + 2, 0)
            wait_gather(1)
            mul_stage(1)
            start_scatter(1)

            @pl.when(c0 + 3 < NCHUNK)
            def _():
                start_loads(c0 + 3, 1)

            return carry

        lax.fori_loop(1, (NCHUNK - 1) // 2, body2, 0)

        wait_loads(0)
        wait_scatter(0)
        start_gather(0)
        wait_gather(0)
        mul_stage(0)
        start_scatter(0)
        wait_scatter(0)
        wait_scatter(1)
        plsc.subcore_barrier()

        rows = pl.ds(rbase, N_TILE)
        pltpu.sync_copy(h_sh.at[rows], out_hbm.at[cid, rows])

        @pl.when(sid == 0)
        def _write_tail():
            tail = pl.ds(N_TILE * NS, N_TAIL)
            pltpu.sync_copy(h_sh.at[tail], out_hbm.at[cid, tail])

    return k(hv, filt, src, dst)


def kernel(x, edge_index, edge_basis, W_pre, b_pre, W_rad, b_rad, W_post, b_post):
    src = edge_index[0]
    dst = edge_index[1]
    hv = _matmul_bias(x, W_pre, b_pre)
    eb8 = edge_basis.reshape(N_EDGES // 8, 8 * D_RADIAL)
    W_big = jnp.zeros((8, D_RADIAL, 8, D), jnp.float32)
    W_big = W_big.at[jnp.arange(8), :, jnp.arange(8), :].set(W_rad)
    W_big = W_big.reshape(8 * D_RADIAL, 8 * D)
    b_big = jnp.tile(b_rad, 8)
    filt = _radial_call(eb8, W_big, b_big)
    parts = _edge_call(hv, filt, src, dst)
    return _post_call(parts, W_post, b_post)
